# baseline (device time: 410641 ns/iter reference)
import jax
import jax.numpy as jnp
from jax import lax
from jax.experimental import pallas as pl
from jax.experimental.pallas import tpu as pltpu

M_SHARD = 16384
N_GLOBAL = 2048
N_HALF = N_GLOBAL // 2
N_CHUNK = 16
R = M_SHARD // N_CHUNK
N_SEND = 4


def kernel(x):
    def body(x_hbm, o_hbm, inp_buf, inl_buf, send_buf, loc_buf,
             inp_sems, inl_sems, out_sems, send_sems, recv_sems):
        my_x = lax.axis_index("x")
        my_y = lax.axis_index("y")
        my_z = lax.axis_index("z")
        peer_y = 1 - my_y
        peer = (my_x, peer_y, my_z)

        my_cols = pl.ds(my_y * N_HALF, N_HALF)
        peer_cols = pl.ds(peer_y * N_HALF, N_HALF)

        def start_loads(i):
            s = i % 2
            rows = pl.ds(i * R, R)
            lp = pltpu.make_async_copy(
                x_hbm.at[rows, peer_cols], inp_buf.at[s], inp_sems.at[s])
            ll = pltpu.make_async_copy(
                x_hbm.at[rows, my_cols], inl_buf.at[s], inl_sems.at[s])
            lp.start()
            ll.start()
            return lp, ll

        loads = [start_loads(0), start_loads(1)]

        barrier = pltpu.get_barrier_semaphore()
        pl.semaphore_signal(barrier, inc=1, device_id=peer,
                            device_id_type=pl.DeviceIdType.MESH)
        pl.semaphore_wait(barrier, 1)

        rdmas = [None] * N_SEND
        stores = [None] * 2
        for i in range(N_CHUNK):
            s = i % 2
            ss = i % N_SEND
            lp, ll = loads[s]
            lp.wait()
            ll.wait()
            if rdmas[ss] is not None:
                rdmas[ss].wait_send()
            send_buf[ss, :, :] = inp_buf[s].astype(jnp.bfloat16)
            if stores[s] is not None:
                stores[s].wait()
            loc_buf[s, :, :] = inl_buf[s].astype(jnp.bfloat16)
            if i + 2 < N_CHUNK:
                loads[s] = start_loads(i + 2)

            dst_rows = pl.ds(my_y * M_SHARD + i * R, R)
            rdma = pltpu.make_async_remote_copy(
                src_ref=send_buf.at[ss],
                dst_ref=o_hbm.at[dst_rows, :],
                send_sem=send_sems.at[ss],
                recv_sem=recv_sems.at[i],
                device_id=peer,
                device_id_type=pl.DeviceIdType.MESH,
            )
            rdma.start()
            rdmas[ss] = rdma

            store = pltpu.make_async_copy(
                loc_buf.at[s], o_hbm.at[dst_rows, :], out_sems.at[s])
            store.start()
            stores[s] = store

        for r_ in rdmas:
            r_.wait_send()
        for st in stores:
            st.wait()

        for i in range(N_CHUNK):
            recv_rows = pl.ds(peer_y * M_SHARD + i * R, R)
            recv = pltpu.make_async_remote_copy(
                src_ref=send_buf.at[0],
                dst_ref=o_hbm.at[recv_rows, :],
                send_sem=send_sems.at[0],
                recv_sem=recv_sems.at[i],
                device_id=peer,
                device_id_type=pl.DeviceIdType.MESH,
            )
            recv.wait_recv()

    out_shape = jax.ShapeDtypeStruct((2 * M_SHARD, N_HALF), jnp.bfloat16)
    return pl.pallas_call(
        body,
        out_shape=out_shape,
        in_specs=[pl.BlockSpec(memory_space=pl.ANY)],
        out_specs=pl.BlockSpec(memory_space=pl.ANY),
        scratch_shapes=[
            pltpu.VMEM((2, R, N_HALF), jnp.float32),
            pltpu.VMEM((2, R, N_HALF), jnp.float32),
            pltpu.VMEM((N_SEND, R, N_HALF), jnp.bfloat16),
            pltpu.VMEM((2, R, N_HALF), jnp.bfloat16),
            pltpu.SemaphoreType.DMA((2,)),
            pltpu.SemaphoreType.DMA((2,)),
            pltpu.SemaphoreType.DMA((2,)),
            pltpu.SemaphoreType.DMA((N_SEND,)),
            pltpu.SemaphoreType.DMA((N_CHUNK,)),
        ],
        compiler_params=pltpu.CompilerParams(
            collective_id=0, vmem_limit_bytes=48 * 1024 * 1024),
    )(x)


# device time: 366336 ns/iter; 1.1209x vs baseline; 1.1209x over previous
import jax
import jax.numpy as jnp
from jax import lax
from jax.experimental import pallas as pl
from jax.experimental.pallas import tpu as pltpu

M_SHARD = 16384
N_GLOBAL = 2048
N_HALF = N_GLOBAL // 2
N_CHUNK = 8
R = M_SHARD // N_CHUNK
N_SEND = 2


def kernel(x):
    def body(x_hbm, o_hbm, inp_buf, inl_buf, send_buf, loc_buf,
             inp_sems, inl_sems, out_sems, send_sems, recv_sems):
        my_x = lax.axis_index("x")
        my_y = lax.axis_index("y")
        my_z = lax.axis_index("z")
        peer_y = 1 - my_y
        peer = (my_x, peer_y, my_z)

        my_cols = pl.ds(my_y * N_HALF, N_HALF)
        peer_cols = pl.ds(peer_y * N_HALF, N_HALF)

        def start_loads(i):
            s = i % 2
            rows = pl.ds(i * R, R)
            lp = pltpu.make_async_copy(
                x_hbm.at[rows, peer_cols], inp_buf.at[s], inp_sems.at[s])
            ll = pltpu.make_async_copy(
                x_hbm.at[rows, my_cols], inl_buf.at[s], inl_sems.at[s])
            lp.start()
            ll.start()
            return lp, ll

        loads = [start_loads(0), start_loads(1)]

        barrier = pltpu.get_barrier_semaphore()
        pl.semaphore_signal(barrier, inc=1, device_id=peer,
                            device_id_type=pl.DeviceIdType.MESH)
        pl.semaphore_wait(barrier, 1)

        rdmas = [None] * N_SEND
        stores = [None] * 2
        for i in range(N_CHUNK):
            s = i % 2
            ss = i % N_SEND
            lp, ll = loads[s]
            lp.wait()
            ll.wait()
            if rdmas[ss] is not None:
                rdmas[ss].wait_send()
            send_buf[ss, :, :] = inp_buf[s].astype(jnp.bfloat16)
            if stores[s] is not None:
                stores[s].wait()
            loc_buf[s, :, :] = inl_buf[s].astype(jnp.bfloat16)
            if i + 2 < N_CHUNK:
                loads[s] = start_loads(i + 2)

            dst_rows = pl.ds(my_y * M_SHARD + i * R, R)
            rdma = pltpu.make_async_remote_copy(
                src_ref=send_buf.at[ss],
                dst_ref=o_hbm.at[dst_rows, :],
                send_sem=send_sems.at[ss],
                recv_sem=recv_sems.at[i],
                device_id=peer,
                device_id_type=pl.DeviceIdType.MESH,
            )
            rdma.start()
            rdmas[ss] = rdma

            store = pltpu.make_async_copy(
                loc_buf.at[s], o_hbm.at[dst_rows, :], out_sems.at[s])
            store.start()
            stores[s] = store

        for r_ in rdmas:
            r_.wait_send()
        for st in stores:
            st.wait()

        for i in range(N_CHUNK):
            recv_rows = pl.ds(peer_y * M_SHARD + i * R, R)
            recv = pltpu.make_async_remote_copy(
                src_ref=send_buf.at[0],
                dst_ref=o_hbm.at[recv_rows, :],
                send_sem=send_sems.at[0],
                recv_sem=recv_sems.at[i],
                device_id=peer,
                device_id_type=pl.DeviceIdType.MESH,
            )
            recv.wait_recv()

    out_shape = jax.ShapeDtypeStruct((2 * M_SHARD, N_HALF), jnp.bfloat16)
    return pl.pallas_call(
        body,
        out_shape=out_shape,
        in_specs=[pl.BlockSpec(memory_space=pl.ANY)],
        out_specs=pl.BlockSpec(memory_space=pl.ANY),
        scratch_shapes=[
            pltpu.VMEM((2, R, N_HALF), jnp.float32),
            pltpu.VMEM((2, R, N_HALF), jnp.float32),
            pltpu.VMEM((N_SEND, R, N_HALF), jnp.bfloat16),
            pltpu.VMEM((2, R, N_HALF), jnp.bfloat16),
            pltpu.SemaphoreType.DMA((2,)),
            pltpu.SemaphoreType.DMA((2,)),
            pltpu.SemaphoreType.DMA((2,)),
            pltpu.SemaphoreType.DMA((N_SEND,)),
            pltpu.SemaphoreType.DMA((N_CHUNK,)),
        ],
        compiler_params=pltpu.CompilerParams(
            collective_id=0, vmem_limit_bytes=60 * 1024 * 1024),
    )(x)
